# baseline (speedup 1.0000x reference)
import functools

import jax
import jax.numpy as jnp
from jax import lax
from jax.experimental import pallas as pl
from jax.experimental.pallas import tpu as pltpu
from jax.experimental.pallas import tpu_sc as plsc

NC, NS = 2, 16
NW = NC * NS
B, S, E = 16384, 200, 64
HID, NCLS = 128, 1000
G0 = 128
G1 = S - G0
CH = 64
NBUF = 4
B_PER_W = B // NW


def _sc_embed_sum(inputs, table):
    mesh = plsc.VectorSubcoreMesh(core_axis_name="c", subcore_axis_name="s")

    @functools.partial(
        pl.kernel,
        out_type=jax.ShapeDtypeStruct((B * E,), jnp.float32),
        mesh=mesh,
        scratch_types=[
            pltpu.VMEM((CH, S), jnp.int32),
            [pltpu.VMEM((S, E), jnp.float32) for _ in range(NBUF)],
            pltpu.VMEM((CH * E,), jnp.float32),
            [pltpu.SemaphoreType.DMA for _ in range(NBUF)],
        ],
        compiler_params=pltpu.CompilerParams(use_tc_tiling_on_sc=False),
    )
    def k(table_hbm, idx_hbm, out_hbm, idx_v, bufs, out_v, sems):
        wid = lax.axis_index("s") * NC + lax.axis_index("c")
        base = wid * B_PER_W

        def issue(i, buf, sem):
            pltpu.async_copy(
                table_hbm.at[idx_v.at[i, pl.ds(0, G0)]],
                buf.at[pl.ds(0, G0)], sem)
            pltpu.async_copy(
                table_hbm.at[idx_v.at[i, pl.ds(G0, G1)]],
                buf.at[pl.ds(G0, G1)], sem)

        def drain(buf, sem):
            pltpu.make_async_copy(table_hbm.at[pl.ds(0, S)], buf, sem).wait()

        def accum(buf, i):
            z = jnp.zeros((16,), jnp.float32)

            def body(r, acc):
                return tuple(
                    acc[j] + buf[r, 16 * j:16 * (j + 1)] for j in range(4))

            acc = lax.fori_loop(0, S, body, (z, z, z, z), unroll=4)
            ob = pl.multiple_of(i * E, 8)
            for j in range(4):
                out_v[pl.ds(ob + 16 * j, 16)] = acc[j]

        @pl.loop(0, B_PER_W, step=CH)
        def _(r0):
            pltpu.sync_copy(idx_hbm.at[pl.ds(base + r0, CH)], idx_v)
            for b in range(NBUF):
                issue(b, bufs[b], sems[b])

            @pl.loop(0, CH, step=NBUF)
            def _(i):
                for b in range(NBUF):
                    drain(bufs[b], sems[b])

                    @pl.when(i + NBUF + b < CH)
                    def _():
                        issue(i + NBUF + b, bufs[b], sems[b])

                    accum(bufs[b], i + b)

            oo = pl.multiple_of((base + r0) * E, 8)
            pltpu.sync_copy(out_v, out_hbm.at[pl.ds(oo, CH * E)])

    return k(table, inputs)


def _tc_mlp(summed_flat, W1, b1, W2, b2):
    BLK = 1024
    NP = 1024

    W1big = jnp.zeros((128, 2 * HID), jnp.float32)
    W1big = W1big.at[0:E, 0:HID].set(W1.T)
    W1big = W1big.at[E:2 * E, HID:2 * HID].set(W1.T)
    b1big = jnp.concatenate([b1, b1]).reshape(1, 2 * HID)
    W2p = jnp.zeros((HID, NP), jnp.float32).at[:, 0:NCLS].set(W2.T)
    W2big = jnp.zeros((2 * HID, 2 * NP), jnp.float32)
    W2big = W2big.at[0:HID, 0:NP].set(W2p)
    W2big = W2big.at[HID:2 * HID, NP:2 * NP].set(W2p)
    b2p = jnp.zeros((NP,), jnp.float32).at[0:NCLS].set(b2)
    b2big = jnp.concatenate([b2p, b2p]).reshape(1, 2 * NP)

    def body(x_ref, w1_ref, b1_ref, w2_ref, b2_ref, o_ref):
        xp = jnp.tanh(x_ref[...].reshape(BLK // 2, 128))
        h = lax.dot_general(
            xp, w1_ref[...], (((1,), (0,)), ((), ())),
            preferred_element_type=jnp.float32,
            precision=lax.Precision.HIGHEST)
        h = jnp.tanh(h + b1_ref[...])
        o = lax.dot_general(
            h, w2_ref[...], (((1,), (0,)), ((), ())),
            preferred_element_type=jnp.float32,
            precision=lax.Precision.HIGHEST)
        o = jnp.tanh(o + b2_ref[...])
        o_ref[...] = o.reshape(BLK, NP)[:, 0:NCLS]

    return pl.pallas_call(
        body,
        grid=(B // BLK,),
        in_specs=[
            pl.BlockSpec((BLK * E,), lambda i: (i,)),
            pl.BlockSpec((128, 2 * HID), lambda i: (0, 0)),
            pl.BlockSpec((1, 2 * HID), lambda i: (0, 0)),
            pl.BlockSpec((2 * HID, 2 * NP), lambda i: (0, 0)),
            pl.BlockSpec((1, 2 * NP), lambda i: (0, 0)),
        ],
        out_specs=pl.BlockSpec((BLK, NCLS), lambda i: (i, 0)),
        out_shape=jax.ShapeDtypeStruct((B, NCLS), jnp.float32),
    )(summed_flat, W1big, b1big, W2big, b2big)


def kernel(inputs, table, W1, b1, W2, b2):
    summed = _sc_embed_sum(inputs, table)
    return _tc_mlp(summed, W1, b1, W2, b2)


# speedup vs baseline: 1.0126x; 1.0126x over previous
import functools

import jax
import jax.numpy as jnp
from jax import lax
from jax.experimental import pallas as pl
from jax.experimental.pallas import tpu as pltpu
from jax.experimental.pallas import tpu_sc as plsc

NC, NS = 2, 16
NW = NC * NS
B, S, E = 16384, 200, 64
HID, NCLS = 128, 1000
G0 = 128
G1 = S - G0
CH = 128
NBUF = 4
B_PER_W = B // NW


def _sc_embed_sum(inputs, table):
    mesh = plsc.VectorSubcoreMesh(core_axis_name="c", subcore_axis_name="s")

    @functools.partial(
        pl.kernel,
        out_type=jax.ShapeDtypeStruct((B, E), jnp.float32),
        mesh=mesh,
        scratch_types=[
            pltpu.VMEM((CH, S), jnp.int32),
            [pltpu.VMEM((S, E), jnp.float32) for _ in range(NBUF)],
            pltpu.VMEM((CH, E), jnp.float32),
            [pltpu.SemaphoreType.DMA for _ in range(NBUF)],
        ],
        compiler_params=pltpu.CompilerParams(use_tc_tiling_on_sc=False),
    )
    def k(table_hbm, idx_hbm, out_hbm, idx_v, bufs, out_v, sems):
        wid = lax.axis_index("s") * NC + lax.axis_index("c")
        base = wid * B_PER_W

        def issue(i, buf, sem):
            pltpu.async_copy(
                table_hbm.at[idx_v.at[i, pl.ds(0, G0)]],
                buf.at[pl.ds(0, G0)], sem)
            pltpu.async_copy(
                table_hbm.at[idx_v.at[i, pl.ds(G0, G1)]],
                buf.at[pl.ds(G0, G1)], sem)

        def drain(buf, sem):
            pltpu.make_async_copy(table_hbm.at[pl.ds(0, S)], buf, sem).wait()

        def accum(buf, i):
            z = jnp.zeros((16,), jnp.float32)

            def body(r, acc):
                return tuple(
                    acc[j] + buf[r, 16 * j:16 * (j + 1)] for j in range(4))

            acc = lax.fori_loop(0, S, body, (z, z, z, z), unroll=4)
            for j in range(4):
                out_v[i, 16 * j:16 * (j + 1)] = acc[j]

        @pl.loop(0, B_PER_W, step=CH)
        def _(r0):
            pltpu.sync_copy(idx_hbm.at[pl.ds(base + r0, CH)], idx_v)
            for b in range(NBUF):
                issue(b, bufs[b], sems[b])

            @pl.loop(0, CH, step=NBUF)
            def _(i):
                for b in range(NBUF):
                    drain(bufs[b], sems[b])

                    @pl.when(i + NBUF + b < CH)
                    def _():
                        issue(i + NBUF + b, bufs[b], sems[b])

                    accum(bufs[b], i + b)

            pltpu.sync_copy(out_v, out_hbm.at[pl.ds(base + r0, CH)])

    return k(table, inputs)


def _tc_mlp(summed, W1, b1, W2, b2):
    BLK = 1024

    def body(x_ref, w1_ref, b1_ref, w2_ref, b2_ref, o_ref):
        x = jnp.tanh(x_ref[...])
        h = lax.dot_general(
            x, w1_ref[...], (((1,), (1,)), ((), ())),
            preferred_element_type=jnp.float32,
            precision=lax.Precision.HIGHEST)
        h = jnp.tanh(h + b1_ref[...])
        o = lax.dot_general(
            h, w2_ref[...], (((1,), (1,)), ((), ())),
            preferred_element_type=jnp.float32,
            precision=lax.Precision.HIGHEST)
        o_ref[...] = jnp.tanh(o + b2_ref[...])

    return pl.pallas_call(
        body,
        grid=(B // BLK,),
        in_specs=[
            pl.BlockSpec((BLK, E), lambda i: (i, 0)),
            pl.BlockSpec((HID, E), lambda i: (0, 0)),
            pl.BlockSpec((1, HID), lambda i: (0, 0)),
            pl.BlockSpec((NCLS, HID), lambda i: (0, 0)),
            pl.BlockSpec((1, NCLS), lambda i: (0, 0)),
        ],
        out_specs=pl.BlockSpec((BLK, NCLS), lambda i: (i, 0)),
        out_shape=jax.ShapeDtypeStruct((B, NCLS), jnp.float32),
    )(summed, W1, b1.reshape(1, HID), W2, b2.reshape(1, NCLS))


def kernel(inputs, table, W1, b1, W2, b2):
    summed = _sc_embed_sum(inputs, table)
    return _tc_mlp(summed, W1, b1, W2, b2)
